# Initial kernel scaffold; baseline (speedup 1.0000x reference)
import functools

import jax
import jax.numpy as jnp
from jax import lax
from jax.experimental import pallas as pl
from jax.experimental.pallas import tpu as pltpu
from jax.experimental.pallas import tpu_sc as plsc

_N = 10000
_NPAD = 10240
_E = 320000
_NC = 2
_NS = 16
_CHUNK = 128
_CPT = 79
_EPAD = _NC * _NS * _CPT * _CHUNK
_RPT = _NPAD // _NS

_mesh = plsc.VectorSubcoreMesh(core_axis_name="c", subcore_axis_name="s")


def _make_sc_agg(feat):
    @functools.partial(
        pl.kernel,
        out_type=(
            jax.ShapeDtypeStruct((_NPAD, feat), jnp.float32),
            jax.ShapeDtypeStruct((_NPAD, feat), jnp.float32),
        ),
        mesh=_mesh,
        scratch_types=[
            pltpu.VMEM((_CHUNK,), jnp.int32),
            pltpu.VMEM((_CHUNK,), jnp.int32),
            pltpu.VMEM((_CHUNK, feat), jnp.float32),
            pltpu.VMEM_SHARED((_NPAD, feat), jnp.float32),
            pltpu.SemaphoreType.DMA,
        ],
    )
    def agg(p_hbm, src_hbm, dst_hbm, zrows_hbm, out0, out1,
            sidx, didx, rows, acc, sem):
        cid = lax.axis_index("c")
        sid = lax.axis_index("s")
        wid = sid * _NC + cid
        r0 = pl.multiple_of(sid * _RPT, _RPT)
        pltpu.sync_copy(zrows_hbm.at[pl.ds(0, _RPT)], acc.at[pl.ds(r0, _RPT)])
        plsc.subcore_barrier()

        def body(g, _):
            base = pl.multiple_of((wid * _CPT + g) * _CHUNK, _CHUNK)
            pltpu.sync_copy(src_hbm.at[pl.ds(base, _CHUNK)], sidx)
            pltpu.sync_copy(dst_hbm.at[pl.ds(base, _CHUNK)], didx)
            pltpu.async_copy(p_hbm.at[sidx], rows, sem).wait()
            pltpu.sync_copy(rows, acc.at[didx], add=True)
            return ()

        lax.fori_loop(0, _CPT, body, ())
        plsc.subcore_barrier()

        @pl.when(cid == 0)
        def _():
            pltpu.sync_copy(acc.at[pl.ds(r0, _RPT)], out0.at[pl.ds(r0, _RPT)])

        @pl.when(cid == 1)
        def _():
            pltpu.sync_copy(acc.at[pl.ds(r0, _RPT)], out1.at[pl.ds(r0, _RPT)])

    return agg


@functools.partial(
    pl.kernel,
    out_type=(
        jax.ShapeDtypeStruct((_NPAD, 16), jnp.float32),
        jax.ShapeDtypeStruct((_NPAD, 16), jnp.float32),
    ),
    mesh=_mesh,
    scratch_types=[
        pltpu.VMEM((_CHUNK,), jnp.int32),
        pltpu.VMEM((_CHUNK, 16), jnp.float32),
        pltpu.VMEM_SHARED((_NPAD, 16), jnp.float32),
    ],
)
def _sc_degree(dst_hbm, ones_hbm, zrows_hbm, out0, out1, didx, rows, acc):
    cid = lax.axis_index("c")
    sid = lax.axis_index("s")
    wid = sid * _NC + cid
    r0 = pl.multiple_of(sid * _RPT, _RPT)
    pltpu.sync_copy(zrows_hbm.at[pl.ds(0, _RPT)], acc.at[pl.ds(r0, _RPT)])
    pltpu.sync_copy(ones_hbm, rows)
    plsc.subcore_barrier()

    def body(g, _):
        base = pl.multiple_of((wid * _CPT + g) * _CHUNK, _CHUNK)
        pltpu.sync_copy(dst_hbm.at[pl.ds(base, _CHUNK)], didx)
        pltpu.sync_copy(rows, acc.at[didx], add=True)
        return ()

    lax.fori_loop(0, _CPT, body, ())
    plsc.subcore_barrier()

    @pl.when(cid == 0)
    def _():
        pltpu.sync_copy(acc.at[pl.ds(r0, _RPT)], out0.at[pl.ds(r0, _RPT)])

    @pl.when(cid == 1)
    def _():
        pltpu.sync_copy(acc.at[pl.ds(r0, _RPT)], out1.at[pl.ds(r0, _RPT)])


_BLK = 256
_GRID = _NPAD // _BLK


def _tc_p1_dis(x_pad, W1, dega, degb):
    def body(x_ref, w_ref, da_ref, db_ref, p_ref, dis_ref):
        i = pl.program_id(0)
        deg = da_ref[...][:, 0:1] + db_ref[...][:, 0:1] + 1.0
        row = i * _BLK + lax.broadcasted_iota(jnp.int32, (_BLK, 1), 0)
        dis = jnp.where(row < _N, lax.rsqrt(deg), 0.0)
        p = jnp.dot(x_ref[...], w_ref[...], preferred_element_type=jnp.float32)
        p_ref[...] = p * dis
        dis_ref[...] = jnp.broadcast_to(dis, (_BLK, 16))

    return pl.pallas_call(
        body,
        grid=(_GRID,),
        in_specs=[
            pl.BlockSpec((_BLK, 128), lambda i: (i, 0)),
            pl.BlockSpec((128, 16), lambda i: (0, 0)),
            pl.BlockSpec((_BLK, 16), lambda i: (i, 0)),
            pl.BlockSpec((_BLK, 16), lambda i: (i, 0)),
        ],
        out_specs=[
            pl.BlockSpec((_BLK, 16), lambda i: (i, 0)),
            pl.BlockSpec((_BLK, 16), lambda i: (i, 0)),
        ],
        out_shape=[
            jax.ShapeDtypeStruct((_NPAD, 16), jnp.float32),
            jax.ShapeDtypeStruct((_NPAD, 16), jnp.float32),
        ],
    )(x_pad, W1, dega, degb)


def _tc_next_p(sa, sb, p, dis, W, b, relu, fin):
    fout = W.shape[1]

    def body(sa_ref, sb_ref, p_ref, dis_ref, w_ref, b_ref, out_ref):
        d = dis_ref[...][:, 0:1]
        h = d * (sa_ref[...] + sb_ref[...] + p_ref[...]) + b_ref[...]
        if relu:
            h = jnp.maximum(h, 0.0)
        out_ref[...] = jnp.dot(h, w_ref[...],
                               preferred_element_type=jnp.float32) * d

    return pl.pallas_call(
        body,
        grid=(_GRID,),
        in_specs=[
            pl.BlockSpec((_BLK, fin), lambda i: (i, 0)),
            pl.BlockSpec((_BLK, fin), lambda i: (i, 0)),
            pl.BlockSpec((_BLK, fin), lambda i: (i, 0)),
            pl.BlockSpec((_BLK, 16), lambda i: (i, 0)),
            pl.BlockSpec((fin, fout), lambda i: (0, 0)),
            pl.BlockSpec((1, fin), lambda i: (0, 0)),
        ],
        out_specs=pl.BlockSpec((_BLK, fout), lambda i: (i, 0)),
        out_shape=jax.ShapeDtypeStruct((_NPAD, fout), jnp.float32),
    )(sa, sb, p, dis, W, b)


def _tc_head(sa, sb, p3, dis, b3, gfeat, fc1_W, fc1_b, fc2_W, fc2_b,
             fcg_W, fcg_b, fc3_W, fc3_b, fc4_W, fc4_b, fcf_W, fcf_b):
    def body(sa_ref, sb_ref, p_ref, dis_ref, b3_ref, gf_ref,
             fc1w, fc1b, fc2w, fc2b, fcgw, fcgb, fc3w, fc3b,
             fc4w, fc4b, fcfw, fcfb, out_ref):
        d = dis_ref[...][:, 0:1]
        h3 = d * (sa_ref[...] + sb_ref[...] + p_ref[...]) + b3_ref[...]
        row = lax.broadcasted_iota(jnp.int32, (_NPAD, 1), 0)
        h3 = jnp.where(row < _N, h3, 0.0)
        g = jnp.sum(h3, axis=0, keepdims=True) * (1.0 / _N)
        g = jnp.dot(g, fc1w[...], preferred_element_type=jnp.float32) + fc1b[...]
        g = jnp.dot(g, fc2w[...], preferred_element_type=jnp.float32) + fc2b[...]
        gf = jnp.maximum(jnp.dot(gf_ref[...], fcgw[...],
                                 preferred_element_type=jnp.float32) + fcgb[...], 0.0)
        gf = jnp.maximum(jnp.dot(gf, fc3w[...],
                                 preferred_element_type=jnp.float32) + fc3b[...], 0.0)
        gf = jnp.maximum(jnp.dot(gf, fc4w[...],
                                 preferred_element_type=jnp.float32) + fc4b[...], 0.0)
        top = jnp.dot(g, fcfw[0:128, :], preferred_element_type=jnp.float32)
        bot = jnp.dot(gf, fcfw[128:384, :], preferred_element_type=jnp.float32)
        out_ref[...] = jnp.maximum(top + bot + fcfb[...], 0.0)

    return pl.pallas_call(
        body,
        out_shape=jax.ShapeDtypeStruct((512, 256), jnp.float32),
    )(sa, sb, p3, dis, b3, gfeat, fc1_W, fc1_b, fc2_W, fc2_b,
      fcg_W, fcg_b, fc3_W, fc3_b, fc4_W, fc4_b, fcf_W, fcf_b)


_agg16 = _make_sc_agg(16)
_agg32 = _make_sc_agg(32)


def kernel(x, edge_index, global_features, W1, b1, W2, b2, W3, b3,
           fc1_W, fc1_b, fc2_W, fc2_b, fcg_W, fcg_b, fc3_W, fc3_b,
           fc4_W, fc4_b, fcf_W, fcf_b):
    pad_e = _EPAD - _E
    src = jnp.concatenate([edge_index[0], jnp.full((pad_e,), _N, jnp.int32)])
    dst = jnp.concatenate([edge_index[1], jnp.full((pad_e,), _N, jnp.int32)])
    x_pad = jnp.concatenate(
        [x, jnp.zeros((_NPAD - _N, 128), jnp.float32)], axis=0)
    ones16 = jnp.ones((_CHUNK, 16), jnp.float32)
    z16 = jnp.zeros((_RPT, 16), jnp.float32)
    z32 = jnp.zeros((_RPT, 32), jnp.float32)
    b1r = b1.reshape(1, 16)
    b2r = b2.reshape(1, 32)
    b3r = b3.reshape(1, 32)

    dega, degb = _sc_degree(dst, ones16, z16)
    p1, dis = _tc_p1_dis(x_pad, W1, dega, degb)
    s1a, s1b = _agg16(p1, src, dst, z16)
    p2 = _tc_next_p(s1a, s1b, p1, dis, W2, b1r, True, 16)
    s2a, s2b = _agg32(p2, src, dst, z32)
    p3 = _tc_next_p(s2a, s2b, p2, dis, W3, b2r, False, 32)
    s3a, s3b = _agg32(p3, src, dst, z32)
    return _tc_head(s3a, s3b, p3, dis, b3r, global_features,
                    fc1_W, fc1_b.reshape(1, 64), fc2_W, fc2_b.reshape(1, 128),
                    fcg_W, fcg_b.reshape(1, 64), fc3_W, fc3_b.reshape(1, 128),
                    fc4_W, fc4_b.reshape(1, 256), fcf_W, fcf_b.reshape(1, 256))


# speedup vs baseline: 63.0234x; 63.0234x over previous
import functools

import jax
import jax.numpy as jnp
from jax import lax
from jax.experimental import pallas as pl
from jax.experimental.pallas import tpu as pltpu
from jax.experimental.pallas import tpu_sc as plsc

_N = 10000
_NPAD = 10240
_E = 320000
_NC = 2
_NS = 16
_F = 32
_TR = 256
_NT = 40
_EPAD = _NC * _NS * _NT * _TR
_RPT = _NPAD // _NS
_PR = _NPAD * _F // 128
_VR = _N * _F // 128


def _mesh():
    return plsc.VectorSubcoreMesh(core_axis_name="c", subcore_axis_name="s",
                                  num_cores=_NC, num_subcores=_NS)


@functools.cache
def _make_sc_agg(gather_p):
    @functools.partial(
        pl.kernel,
        out_type=(
            jax.ShapeDtypeStruct((_NPAD, _F), jnp.float32),
            jax.ShapeDtypeStruct((_NPAD, _F), jnp.float32),
        ),
        mesh=_mesh(),
        compiler_params=pltpu.CompilerParams(use_tc_tiling_on_sc=False),
        scratch_types=[
            pltpu.VMEM((_NT * _TR,), jnp.int32),
            pltpu.VMEM((_NT * _TR,), jnp.int32),
            pltpu.VMEM((4, _TR, _F), jnp.float32),
            pltpu.VMEM_SHARED((_NPAD, _F), jnp.float32),
        ] + [pltpu.SemaphoreType.DMA] * 8,
    )
    def agg(p_hbm, src_hbm, dst_hbm, zrows_hbm, out0, out1,
            sidx, didx, rows, acc, *sems):
        cid = lax.axis_index("c")
        sid = lax.axis_index("s")
        wid = sid * _NC + cid
        r0 = pl.multiple_of(sid * _RPT, _RPT)
        pltpu.sync_copy(zrows_hbm.at[pl.ds(0, _RPT)], acc.at[pl.ds(r0, _RPT)])
        e0 = pl.multiple_of(wid * (_NT * _TR), _TR)
        if gather_p:
            pltpu.sync_copy(src_hbm.at[pl.ds(e0, _NT * _TR)], sidx)
        else:
            pltpu.sync_copy(src_hbm, rows.at[0])
        pltpu.sync_copy(dst_hbm.at[pl.ds(e0, _NT * _TR)], didx)
        plsc.subcore_barrier()

        gsem = sems[0:4]
        ssem = sems[4:8]

        def gather(c, k):
            pltpu.async_copy(p_hbm.at[sidx.at[pl.ds(c * _TR, _TR)]],
                             rows.at[k], gsem[k])

        def wait_gather(k):
            pltpu.make_async_copy(p_hbm.at[sidx.at[pl.ds(0, _TR)]],
                                  rows.at[k], gsem[k]).wait()

        def scat(c, k):
            kk = k if gather_p else 0
            pltpu.async_copy(rows.at[kk],
                             acc.at[didx.at[pl.ds(c * _TR, _TR)]],
                             ssem[k], add=True)

        def wait_scat(k):
            kk = k if gather_p else 0
            pltpu.make_async_copy(rows.at[kk],
                                  acc.at[didx.at[pl.ds(0, _TR)]],
                                  ssem[k]).wait()

        niter = _NT // 4
        if gather_p:
            gather(0, 0)
            gather(1, 1)

        def body(j, _):
            c0 = 4 * j
            for k in range(4):
                if gather_p:
                    wait_gather(k)
                scat(c0 + k, k)
                m = (k + 2) % 4
                if k < 2:
                    @pl.when(j > 0)
                    def _():
                        wait_scat(m)

                    if gather_p:
                        gather(c0 + k + 2, m)
                else:
                    wait_scat(m)

                    if gather_p:
                        @pl.when(j < niter - 1)
                        def _():
                            gather(c0 + k + 2, m)
            return ()

        lax.fori_loop(0, niter, body, ())
        wait_scat(2)
        wait_scat(3)
        plsc.subcore_barrier()

        @pl.when(cid == 0)
        def _():
            pltpu.sync_copy(acc.at[pl.ds(r0, _RPT)], out0.at[pl.ds(r0, _RPT)])

        @pl.when(cid == 1)
        def _():
            pltpu.sync_copy(acc.at[pl.ds(r0, _RPT)], out1.at[pl.ds(r0, _RPT)])

    return agg


def _tc_p1_dis(x4, W1bd, dega, degb):
    def body(x_ref, w_ref, da_ref, db_ref, p_ref, dis_ref):
        deg = da_ref[...] + db_ref[...] + 1.0
        row = lax.broadcasted_iota(jnp.int32, (_PR, 1), 0)
        dis = jnp.where(row < _VR, lax.rsqrt(deg), 0.0)
        p = jnp.dot(x_ref[...], w_ref[...], preferred_element_type=jnp.float32)
        p_ref[...] = p * dis
        dis_ref[...] = dis

    return pl.pallas_call(
        body,
        out_shape=[
            jax.ShapeDtypeStruct((_PR, 128), jnp.float32),
            jax.ShapeDtypeStruct((_PR, 128), jnp.float32),
        ],
    )(x4, W1bd, dega, degb)


def _tc_next_p(sa, sb, p, dis, Wbd, btile, relu):
    def body(sa_ref, sb_ref, p_ref, dis_ref, w_ref, b_ref, out_ref):
        d = dis_ref[...]
        h = d * (sa_ref[...] + sb_ref[...] + p_ref[...]) + b_ref[...]
        if relu:
            h = jnp.maximum(h, 0.0)
        out_ref[...] = jnp.dot(h, w_ref[...],
                               preferred_element_type=jnp.float32) * d

    return pl.pallas_call(
        body,
        out_shape=jax.ShapeDtypeStruct((_PR, 128), jnp.float32),
    )(sa, sb, p, dis, Wbd, btile)


def _tc_head(sa, sb, p3, dis, b3t, gfeat, fc1_W, fc1_b, fc2_W, fc2_b,
             fcg_W, fcg_b, fc3_W, fc3_b, fc4_W, fc4_b, fcf_W, fcf_b):
    def body(sa_ref, sb_ref, p_ref, dis_ref, b3_ref, gf_ref,
             fc1w, fc1b, fc2w, fc2b, fcgw, fcgb, fc3w, fc3b,
             fc4w, fc4b, fcfw, fcfb, out_ref):
        d = dis_ref[...]
        h3 = d * (sa_ref[...] + sb_ref[...] + p_ref[...]) + b3_ref[...]
        row = lax.broadcasted_iota(jnp.int32, (_PR, 1), 0)
        h3 = jnp.where(row < _VR, h3, 0.0)
        cs = jnp.sum(h3, axis=0, keepdims=True)
        g = (cs[:, 0:32] + cs[:, 32:64] + cs[:, 64:96]
             + cs[:, 96:128]) * (1.0 / _N)
        g = jnp.dot(g, fc1w[...], preferred_element_type=jnp.float32) + fc1b[...]
        g = jnp.dot(g, fc2w[...], preferred_element_type=jnp.float32) + fc2b[...]
        gf = jnp.maximum(jnp.dot(gf_ref[...], fcgw[...],
                                 preferred_element_type=jnp.float32) + fcgb[...], 0.0)
        gf = jnp.maximum(jnp.dot(gf, fc3w[...],
                                 preferred_element_type=jnp.float32) + fc3b[...], 0.0)
        gf = jnp.maximum(jnp.dot(gf, fc4w[...],
                                 preferred_element_type=jnp.float32) + fc4b[...], 0.0)
        top = jnp.dot(g, fcfw[0:128, :], preferred_element_type=jnp.float32)
        bot = jnp.dot(gf, fcfw[128:384, :], preferred_element_type=jnp.float32)
        out_ref[...] = jnp.maximum(top + bot + fcfb[...], 0.0)

    return pl.pallas_call(
        body,
        out_shape=jax.ShapeDtypeStruct((512, 256), jnp.float32),
    )(sa, sb, p3, dis, b3t, gfeat, fc1_W, fc1_b, fc2_W, fc2_b,
      fcg_W, fcg_b, fc3_W, fc3_b, fc4_W, fc4_b, fcf_W, fcf_b)


def _packed(a):
    return a.reshape(_PR, 128)


def kernel(x, edge_index, global_features, W1, b1, W2, b2, W3, b3,
           fc1_W, fc1_b, fc2_W, fc2_b, fcg_W, fcg_b, fc3_W, fc3_b,
           fc4_W, fc4_b, fcf_W, fcf_b):
    pad_e = _EPAD - _E
    pad_rows = _N + (jnp.arange(pad_e, dtype=jnp.int32) % (_NPAD - _N))
    src = jnp.concatenate([edge_index[0], pad_rows])
    dst = jnp.concatenate([edge_index[1], pad_rows])
    x4 = jnp.concatenate(
        [x, jnp.zeros((_NPAD - _N, 128), jnp.float32)], axis=0
    ).reshape(_NPAD * 128 // 512, 512)
    ones32 = jnp.ones((_TR, _F), jnp.float32)
    z32 = jnp.zeros((_RPT, _F), jnp.float32)
    eye4 = jnp.eye(4, dtype=jnp.float32)
    W1p = jnp.pad(W1, ((0, 0), (0, 16)))
    W2p = jnp.pad(W2, ((0, 16), (0, 0)))
    W1bd = jnp.kron(eye4, W1p)
    W2bd = jnp.kron(eye4, W2p)
    W3bd = jnp.kron(eye4, W3)
    b1t = jnp.tile(jnp.pad(b1, (0, 16)), 4).reshape(1, 128)
    b2t = jnp.tile(b2, 4).reshape(1, 128)
    b3t = jnp.tile(b3, 4).reshape(1, 128)

    agg = _make_sc_agg(True)
    deg = _make_sc_agg(False)
    dega, degb = deg(ones32, ones32, dst, z32)
    p1, dis = _tc_p1_dis(x4, W1bd, _packed(dega), _packed(degb))
    s1a, s1b = agg(p1.reshape(_NPAD, _F), src, dst, z32)
    p2 = _tc_next_p(_packed(s1a), _packed(s1b), p1, dis, W2bd, b1t, True)
    s2a, s2b = agg(p2.reshape(_NPAD, _F), src, dst, z32)
    p3 = _tc_next_p(_packed(s2a), _packed(s2b), p2, dis, W3bd, b2t, False)
    s3a, s3b = agg(p3.reshape(_NPAD, _F), src, dst, z32)
    return _tc_head(_packed(s3a), _packed(s3b), p3, dis, b3t,
                    global_features,
                    fc1_W, fc1_b.reshape(1, 64), fc2_W, fc2_b.reshape(1, 128),
                    fcg_W, fcg_b.reshape(1, 64), fc3_W, fc3_b.reshape(1, 128),
                    fc4_W, fc4_b.reshape(1, 256), fcf_W, fcf_b.reshape(1, 256))


# speedup vs baseline: 69.1505x; 1.0972x over previous
import functools

import jax
import jax.numpy as jnp
from jax import lax
from jax.experimental import pallas as pl
from jax.experimental.pallas import tpu as pltpu
from jax.experimental.pallas import tpu_sc as plsc

_N = 10000
_NPAD = 10240
_E = 320000
_NC = 2
_NS = 16
_F = 32
_TR = 512
_NT = 20
_EPT = _NT * _TR
_EPAD = _NC * _NS * _NT * _TR
_RPT = _NPAD // _NS
_PR = _NPAD * _F // 128
_VR = _N * _F // 128


def _mesh():
    return plsc.VectorSubcoreMesh(core_axis_name="c", subcore_axis_name="s",
                                  num_cores=_NC, num_subcores=_NS)


@functools.cache
def _make_sc_agg(gather_p):
    @functools.partial(
        pl.kernel,
        out_type=(
            jax.ShapeDtypeStruct((_NPAD, _F), jnp.float32),
            jax.ShapeDtypeStruct((_NPAD, _F), jnp.float32),
        ),
        mesh=_mesh(),
        compiler_params=pltpu.CompilerParams(use_tc_tiling_on_sc=False),
        scratch_types=[
            pltpu.VMEM((_NT * _TR,), jnp.int32),
            pltpu.VMEM((_NT * _TR,), jnp.int32),
            pltpu.VMEM((4, _TR, _F), jnp.float32),
            pltpu.VMEM_SHARED((_NPAD, _F), jnp.float32),
        ] + [pltpu.SemaphoreType.DMA] * 8,
    )
    def agg(p_hbm, ei_hbm, pad_hbm, zrows_hbm, out0, out1,
            sidx, didx, rows, acc, *sems):
        cid = lax.axis_index("c")
        sid = lax.axis_index("s")
        wid = sid * _NC + cid
        r0 = pl.multiple_of(sid * _RPT, _RPT)
        pltpu.sync_copy(zrows_hbm.at[pl.ds(0, _RPT)], acc.at[pl.ds(r0, _RPT)])
        e0 = pl.multiple_of(wid * _EPT, _TR)
        real = _E - (_NC * _NS - 1) * _EPT
        last = _NC * _NS - 1

        @pl.when(wid < last)
        def _():
            if gather_p:
                pltpu.sync_copy(ei_hbm.at[pl.ds(e0, _EPT)], sidx)
            pltpu.sync_copy(ei_hbm.at[pl.ds(_E + e0, _EPT)], didx)

        @pl.when(wid == last)
        def _():
            if gather_p:
                pltpu.sync_copy(ei_hbm.at[pl.ds(last * _EPT, real)],
                                sidx.at[pl.ds(0, real)])
                pltpu.sync_copy(pad_hbm, sidx.at[pl.ds(real, _EPT - real)])
            pltpu.sync_copy(ei_hbm.at[pl.ds(_E + last * _EPT, real)],
                            didx.at[pl.ds(0, real)])
            pltpu.sync_copy(pad_hbm, didx.at[pl.ds(real, _EPT - real)])

        if not gather_p:
            pltpu.sync_copy(p_hbm, rows.at[0])
        plsc.subcore_barrier()

        gsem = sems[0:4]
        ssem = sems[4:8]

        def gather(c, k):
            pltpu.async_copy(p_hbm.at[sidx.at[pl.ds(c * _TR, _TR)]],
                             rows.at[k], gsem[k])

        def wait_gather(k):
            pltpu.make_async_copy(p_hbm.at[sidx.at[pl.ds(0, _TR)]],
                                  rows.at[k], gsem[k]).wait()

        def scat(c, k):
            kk = k if gather_p else 0
            pltpu.async_copy(rows.at[kk],
                             acc.at[didx.at[pl.ds(c * _TR, _TR)]],
                             ssem[k], add=True)

        def wait_scat(k):
            kk = k if gather_p else 0
            pltpu.make_async_copy(rows.at[kk],
                                  acc.at[didx.at[pl.ds(0, _TR)]],
                                  ssem[k]).wait()

        niter = _NT // 4
        if gather_p:
            gather(0, 0)
            gather(1, 1)

        def body(j, _):
            c0 = 4 * j
            for k in range(4):
                if gather_p:
                    wait_gather(k)
                scat(c0 + k, k)
                m = (k + 2) % 4
                if k < 2:
                    @pl.when(j > 0)
                    def _():
                        wait_scat(m)

                    if gather_p:
                        gather(c0 + k + 2, m)
                else:
                    wait_scat(m)

                    if gather_p:
                        @pl.when(j < niter - 1)
                        def _():
                            gather(c0 + k + 2, m)
            return ()

        lax.fori_loop(0, niter, body, ())
        wait_scat(2)
        wait_scat(3)
        plsc.subcore_barrier()

        @pl.when(cid == 0)
        def _():
            pltpu.sync_copy(acc.at[pl.ds(r0, _RPT)], out0.at[pl.ds(r0, _RPT)])

        @pl.when(cid == 1)
        def _():
            pltpu.sync_copy(acc.at[pl.ds(r0, _RPT)], out1.at[pl.ds(r0, _RPT)])

    return agg


def _tc_p1_dis(x4, W1bd, dega, degb):
    def body(x_ref, w_ref, da_ref, db_ref, p_ref, dis_ref):
        deg = da_ref[...] + db_ref[...] + 1.0
        row = lax.broadcasted_iota(jnp.int32, (_PR, 1), 0)
        dis = jnp.where(row < _VR, lax.rsqrt(deg), 0.0)
        p = jnp.dot(x_ref[...], w_ref[...], preferred_element_type=jnp.float32)
        p_ref[...] = p * dis
        dis_ref[...] = dis

    return pl.pallas_call(
        body,
        out_shape=[
            jax.ShapeDtypeStruct((_PR, 128), jnp.float32),
            jax.ShapeDtypeStruct((_PR, 128), jnp.float32),
        ],
    )(x4, W1bd, dega, degb)


def _tc_next_p(sa, sb, p, dis, Wbd, btile, relu):
    def body(sa_ref, sb_ref, p_ref, dis_ref, w_ref, b_ref, out_ref):
        d = dis_ref[...]
        h = d * (sa_ref[...] + sb_ref[...] + p_ref[...]) + b_ref[...]
        if relu:
            h = jnp.maximum(h, 0.0)
        out_ref[...] = jnp.dot(h, w_ref[...],
                               preferred_element_type=jnp.float32) * d

    return pl.pallas_call(
        body,
        out_shape=jax.ShapeDtypeStruct((_PR, 128), jnp.float32),
    )(sa, sb, p, dis, Wbd, btile)


def _tc_head(sa, sb, p3, dis, b3t, gfeat, fc1_W, fc1_b, fc2_W, fc2_b,
             fcg_W, fcg_b, fc3_W, fc3_b, fc4_W, fc4_b, fcf_W, fcf_b):
    def body(sa_ref, sb_ref, p_ref, dis_ref, b3_ref, gf_ref,
             fc1w, fc1b, fc2w, fc2b, fcgw, fcgb, fc3w, fc3b,
             fc4w, fc4b, fcfw, fcfb, out_ref):
        d = dis_ref[...]
        h3 = d * (sa_ref[...] + sb_ref[...] + p_ref[...]) + b3_ref[...]
        row = lax.broadcasted_iota(jnp.int32, (_PR, 1), 0)
        h3 = jnp.where(row < _VR, h3, 0.0)
        cs = jnp.sum(h3, axis=0, keepdims=True)
        g = (cs[:, 0:32] + cs[:, 32:64] + cs[:, 64:96]
             + cs[:, 96:128]) * (1.0 / _N)
        g = jnp.dot(g, fc1w[...], preferred_element_type=jnp.float32) + fc1b[...]
        g = jnp.dot(g, fc2w[...], preferred_element_type=jnp.float32) + fc2b[...]
        gf = jnp.maximum(jnp.dot(gf_ref[...], fcgw[...],
                                 preferred_element_type=jnp.float32) + fcgb[...], 0.0)
        gf = jnp.maximum(jnp.dot(gf, fc3w[...],
                                 preferred_element_type=jnp.float32) + fc3b[...], 0.0)
        gf = jnp.maximum(jnp.dot(gf, fc4w[...],
                                 preferred_element_type=jnp.float32) + fc4b[...], 0.0)
        top = jnp.dot(g, fcfw[0:128, :], preferred_element_type=jnp.float32)
        bot = jnp.dot(gf, fcfw[128:384, :], preferred_element_type=jnp.float32)
        out_ref[...] = jnp.maximum(top + bot + fcfb[...], 0.0)

    return pl.pallas_call(
        body,
        out_shape=jax.ShapeDtypeStruct((512, 256), jnp.float32),
    )(sa, sb, p3, dis, b3t, gfeat, fc1_W, fc1_b, fc2_W, fc2_b,
      fcg_W, fcg_b, fc3_W, fc3_b, fc4_W, fc4_b, fcf_W, fcf_b)


def _packed(a):
    return a.reshape(_PR, 128)


def kernel(x, edge_index, global_features, W1, b1, W2, b2, W3, b3,
           fc1_W, fc1_b, fc2_W, fc2_b, fcg_W, fcg_b, fc3_W, fc3_b,
           fc4_W, fc4_b, fcf_W, fcf_b):
    pad_e = _EPT - (_E - (_NC * _NS - 1) * _EPT)
    pad_rows = _N + (jnp.arange(pad_e, dtype=jnp.int32) % (_NPAD - _N))
    ei = edge_index.reshape(2 * _E)
    x4 = jnp.concatenate(
        [x, jnp.zeros((_NPAD - _N, 128), jnp.float32)], axis=0
    ).reshape(_NPAD * 128 // 512, 512)
    ones32 = jnp.ones((_TR, _F), jnp.float32)
    z32 = jnp.zeros((_RPT, _F), jnp.float32)
    eye4 = jnp.eye(4, dtype=jnp.float32)
    W1p = jnp.pad(W1, ((0, 0), (0, 16)))
    W2p = jnp.pad(W2, ((0, 16), (0, 0)))
    W1bd = jnp.kron(eye4, W1p)
    W2bd = jnp.kron(eye4, W2p)
    W3bd = jnp.kron(eye4, W3)
    b1t = jnp.tile(jnp.pad(b1, (0, 16)), 4).reshape(1, 128)
    b2t = jnp.tile(b2, 4).reshape(1, 128)
    b3t = jnp.tile(b3, 4).reshape(1, 128)

    agg = _make_sc_agg(True)
    deg = _make_sc_agg(False)
    dega, degb = deg(ones32, ei, pad_rows, z32)
    p1, dis = _tc_p1_dis(x4, W1bd, _packed(dega), _packed(degb))
    s1a, s1b = agg(p1.reshape(_NPAD, _F), ei, pad_rows, z32)
    p2 = _tc_next_p(_packed(s1a), _packed(s1b), p1, dis, W2bd, b1t, True)
    s2a, s2b = agg(p2.reshape(_NPAD, _F), ei, pad_rows, z32)
    p3 = _tc_next_p(_packed(s2a), _packed(s2b), p2, dis, W3bd, b2t, False)
    s3a, s3b = agg(p3.reshape(_NPAD, _F), ei, pad_rows, z32)
    return _tc_head(_packed(s3a), _packed(s3b), p3, dis, b3t,
                    global_features,
                    fc1_W, fc1_b.reshape(1, 64), fc2_W, fc2_b.reshape(1, 128),
                    fcg_W, fcg_b.reshape(1, 64), fc3_W, fc3_b.reshape(1, 128),
                    fc4_W, fc4_b.reshape(1, 256), fcf_W, fcf_b.reshape(1, 256))


# speedup vs baseline: 72.5772x; 1.0496x over previous
import functools

import jax
import jax.numpy as jnp
from jax import lax
from jax.experimental import pallas as pl
from jax.experimental.pallas import tpu as pltpu
from jax.experimental.pallas import tpu_sc as plsc

_N = 10000
_NPAD = 10240
_E = 320000
_NC = 2
_NS = 16
_F = 32
_TR = 256
_NT = 40
_EPT = _NT * _TR
_NB = 8
_EPAD = _NC * _NS * _NT * _TR
_RPT = _NPAD // _NS
_PR = _NPAD * _F // 128
_VR = _N * _F // 128


def _mesh():
    return plsc.VectorSubcoreMesh(core_axis_name="c", subcore_axis_name="s",
                                  num_cores=_NC, num_subcores=_NS)


@functools.cache
def _make_sc_agg(gather_p):
    fw = _F if gather_p else 16
    scratch = [
        pltpu.VMEM((_EPT,), jnp.int32),
        pltpu.VMEM((_EPT,), jnp.int32),
        pltpu.VMEM((_NB, _TR, fw), jnp.float32),
        pltpu.VMEM_SHARED((_NPAD, fw), jnp.float32),
    ]
    if not gather_p:
        scratch += [pltpu.VMEM((_RPT, 16), jnp.float32),
                    pltpu.VMEM((_RPT, _F), jnp.float32)]
    scratch += [pltpu.SemaphoreType.DMA] * (2 * _NB)

    @functools.partial(
        pl.kernel,
        out_type=(
            jax.ShapeDtypeStruct((_NPAD, _F), jnp.float32),
            jax.ShapeDtypeStruct((_NPAD, _F), jnp.float32),
        ),
        mesh=_mesh(),
        compiler_params=pltpu.CompilerParams(use_tc_tiling_on_sc=False),
        scratch_types=scratch,
    )
    def agg(p_hbm, ei_hbm, pad_hbm, zrows_hbm, out0, out1,
            sidx, didx, rows, acc, *rest):
        if gather_p:
            sems = rest
        else:
            buf16, buf32 = rest[0], rest[1]
            sems = rest[2:]
        gsem = sems[0:_NB]
        ssem = sems[_NB:2 * _NB]
        cid = lax.axis_index("c")
        sid = lax.axis_index("s")
        wid = sid * _NC + cid
        r0 = pl.multiple_of(sid * _RPT, _RPT)
        pltpu.sync_copy(zrows_hbm.at[pl.ds(0, _RPT)], acc.at[pl.ds(r0, _RPT)])
        e0 = pl.multiple_of(wid * _EPT, _TR)
        real = _E - (_NC * _NS - 1) * _EPT
        last = _NC * _NS - 1

        @pl.when(wid < last)
        def _():
            if gather_p:
                pltpu.sync_copy(ei_hbm.at[pl.ds(e0, _EPT)], sidx)
            pltpu.sync_copy(ei_hbm.at[pl.ds(_E + e0, _EPT)], didx)

        @pl.when(wid == last)
        def _():
            if gather_p:
                pltpu.sync_copy(ei_hbm.at[pl.ds(last * _EPT, real)],
                                sidx.at[pl.ds(0, real)])
                pltpu.sync_copy(pad_hbm, sidx.at[pl.ds(real, _EPT - real)])
            pltpu.sync_copy(ei_hbm.at[pl.ds(_E + last * _EPT, real)],
                            didx.at[pl.ds(0, real)])
            pltpu.sync_copy(pad_hbm, didx.at[pl.ds(real, _EPT - real)])

        if not gather_p:
            pltpu.sync_copy(p_hbm, rows.at[0])
        plsc.subcore_barrier()

        def gather(c, k):
            pltpu.async_copy(p_hbm.at[sidx.at[pl.ds(c * _TR, _TR)]],
                             rows.at[k], gsem[k])

        def wait_gather(k):
            pltpu.make_async_copy(p_hbm.at[sidx.at[pl.ds(0, _TR)]],
                                  rows.at[k], gsem[k]).wait()

        def scat(c, k):
            kk = k if gather_p else 0
            pltpu.async_copy(rows.at[kk],
                             acc.at[didx.at[pl.ds(c * _TR, _TR)]],
                             ssem[k], add=True)

        def wait_scat(k):
            kk = k if gather_p else 0
            pltpu.make_async_copy(rows.at[kk],
                                  acc.at[didx.at[pl.ds(0, _TR)]],
                                  ssem[k]).wait()

        half = _NB // 2
        niter = _NT // _NB
        if gather_p:
            for k in range(half):
                gather(k, k)

        def body(j, _):
            c0 = _NB * j
            for k in range(_NB):
                if gather_p:
                    wait_gather(k)
                scat(c0 + k, k)
                m = (k + half) % _NB
                if k < half:
                    @pl.when(j > 0)
                    def _():
                        wait_scat(m)

                    if gather_p:
                        gather(c0 + k + half, m)
                else:
                    wait_scat(m)

                    if gather_p:
                        @pl.when(j < niter - 1)
                        def _():
                            gather(c0 + k + half, m)
            return ()

        lax.fori_loop(0, niter, body, ())
        for k in range(half, _NB):
            wait_scat(k)
        plsc.subcore_barrier()

        if gather_p:
            @pl.when(cid == 0)
            def _():
                pltpu.sync_copy(acc.at[pl.ds(r0, _RPT)],
                                out0.at[pl.ds(r0, _RPT)])

            @pl.when(cid == 1)
            def _():
                pltpu.sync_copy(acc.at[pl.ds(r0, _RPT)],
                                out1.at[pl.ds(r0, _RPT)])
        else:
            pltpu.sync_copy(acc.at[pl.ds(r0, _RPT)], buf16)

            def dup(r, _):
                v = buf16[r, :]
                buf32[r, 0:16] = v
                buf32[r, 16:32] = v
                return ()

            lax.fori_loop(0, _RPT, dup, ())

            @pl.when(cid == 0)
            def _():
                pltpu.sync_copy(buf32, out0.at[pl.ds(r0, _RPT)])

            @pl.when(cid == 1)
            def _():
                pltpu.sync_copy(buf32, out1.at[pl.ds(r0, _RPT)])

    return agg


def _tc_p1_dis(x4, W1bd, dega, degb):
    def body(x_ref, w_ref, da_ref, db_ref, p_ref, dis_ref):
        deg = da_ref[...] + db_ref[...] + 1.0
        row = lax.broadcasted_iota(jnp.int32, (_PR, 1), 0)
        dis = jnp.where(row < _VR, lax.rsqrt(deg), 0.0)
        p = jnp.dot(x_ref[...], w_ref[...], preferred_element_type=jnp.float32)
        p_ref[...] = p * dis
        dis_ref[...] = dis

    return pl.pallas_call(
        body,
        out_shape=[
            jax.ShapeDtypeStruct((_PR, 128), jnp.float32),
            jax.ShapeDtypeStruct((_PR, 128), jnp.float32),
        ],
    )(x4, W1bd, dega, degb)


def _tc_next_p(sa, sb, p, dis, Wbd, btile, relu):
    def body(sa_ref, sb_ref, p_ref, dis_ref, w_ref, b_ref, out_ref):
        d = dis_ref[...]
        h = d * (sa_ref[...] + sb_ref[...] + p_ref[...]) + b_ref[...]
        if relu:
            h = jnp.maximum(h, 0.0)
        out_ref[...] = jnp.dot(h, w_ref[...],
                               preferred_element_type=jnp.float32) * d

    return pl.pallas_call(
        body,
        out_shape=jax.ShapeDtypeStruct((_PR, 128), jnp.float32),
    )(sa, sb, p, dis, Wbd, btile)


def _tc_head(sa, sb, p3, dis, b3t, gfeat, fc1_W, fc1_b, fc2_W, fc2_b,
             fcg_W, fcg_b, fc3_W, fc3_b, fc4_W, fc4_b, fcf_W, fcf_b):
    def body(sa_ref, sb_ref, p_ref, dis_ref, b3_ref, gf_ref,
             fc1w, fc1b, fc2w, fc2b, fcgw, fcgb, fc3w, fc3b,
             fc4w, fc4b, fcfw, fcfb, out_ref):
        d = dis_ref[...]
        h3 = d * (sa_ref[...] + sb_ref[...] + p_ref[...]) + b3_ref[...]
        row = lax.broadcasted_iota(jnp.int32, (_PR, 1), 0)
        h3 = jnp.where(row < _VR, h3, 0.0)
        cs = jnp.sum(h3, axis=0, keepdims=True)
        g = (cs[:, 0:32] + cs[:, 32:64] + cs[:, 64:96]
             + cs[:, 96:128]) * (1.0 / _N)
        g = jnp.dot(g, fc1w[...], preferred_element_type=jnp.float32) + fc1b[...]
        g = jnp.dot(g, fc2w[...], preferred_element_type=jnp.float32) + fc2b[...]
        gf = jnp.maximum(jnp.dot(gf_ref[...], fcgw[...],
                                 preferred_element_type=jnp.float32) + fcgb[...], 0.0)
        gf = jnp.maximum(jnp.dot(gf, fc3w[...],
                                 preferred_element_type=jnp.float32) + fc3b[...], 0.0)
        gf = jnp.maximum(jnp.dot(gf, fc4w[...],
                                 preferred_element_type=jnp.float32) + fc4b[...], 0.0)
        top = jnp.dot(g, fcfw[0:128, :], preferred_element_type=jnp.float32)
        bot = jnp.dot(gf, fcfw[128:384, :], preferred_element_type=jnp.float32)
        out_ref[...] = jnp.maximum(top + bot + fcfb[...], 0.0)

    return pl.pallas_call(
        body,
        out_shape=jax.ShapeDtypeStruct((512, 256), jnp.float32),
    )(sa, sb, p3, dis, b3t, gfeat, fc1_W, fc1_b, fc2_W, fc2_b,
      fcg_W, fcg_b, fc3_W, fc3_b, fc4_W, fc4_b, fcf_W, fcf_b)


def _packed(a):
    return a.reshape(_PR, 128)


def kernel(x, edge_index, global_features, W1, b1, W2, b2, W3, b3,
           fc1_W, fc1_b, fc2_W, fc2_b, fcg_W, fcg_b, fc3_W, fc3_b,
           fc4_W, fc4_b, fcf_W, fcf_b):
    pad_e = _EPT - (_E - (_NC * _NS - 1) * _EPT)
    pad_rows = _N + (jnp.arange(pad_e, dtype=jnp.int32) % (_NPAD - _N))
    ei = edge_index.reshape(2 * _E)
    x4 = jnp.concatenate(
        [x, jnp.zeros((_NPAD - _N, 128), jnp.float32)], axis=0
    ).reshape(_NPAD * 128 // 512, 512)
    ones16 = jnp.ones((_TR, 16), jnp.float32)
    z16 = jnp.zeros((_RPT, 16), jnp.float32)
    z32 = jnp.zeros((_RPT, _F), jnp.float32)
    eye4 = jnp.eye(4, dtype=jnp.float32)
    W1p = jnp.pad(W1, ((0, 0), (0, 16)))
    W2p = jnp.pad(W2, ((0, 16), (0, 0)))
    W1bd = jnp.kron(eye4, W1p)
    W2bd = jnp.kron(eye4, W2p)
    W3bd = jnp.kron(eye4, W3)
    b1t = jnp.tile(jnp.pad(b1, (0, 16)), 4).reshape(1, 128)
    b2t = jnp.tile(b2, 4).reshape(1, 128)
    b3t = jnp.tile(b3, 4).reshape(1, 128)

    agg = _make_sc_agg(True)
    deg = _make_sc_agg(False)
    dega, degb = deg(ones16, ei, pad_rows, z16)
    p1, dis = _tc_p1_dis(x4, W1bd, _packed(dega), _packed(degb))
    s1a, s1b = agg(p1.reshape(_NPAD, _F), ei, pad_rows, z32)
    p2 = _tc_next_p(_packed(s1a), _packed(s1b), p1, dis, W2bd, b1t, True)
    s2a, s2b = agg(p2.reshape(_NPAD, _F), ei, pad_rows, z32)
    p3 = _tc_next_p(_packed(s2a), _packed(s2b), p2, dis, W3bd, b2t, False)
    s3a, s3b = agg(p3.reshape(_NPAD, _F), ei, pad_rows, z32)
    return _tc_head(_packed(s3a), _packed(s3b), p3, dis, b3t,
                    global_features,
                    fc1_W, fc1_b.reshape(1, 64), fc2_W, fc2_b.reshape(1, 128),
                    fcg_W, fcg_b.reshape(1, 64), fc3_W, fc3_b.reshape(1, 128),
                    fc4_W, fc4_b.reshape(1, 256), fcf_W, fcf_b.reshape(1, 256))


# speedup vs baseline: 76.4437x; 1.0533x over previous
import functools

import jax
import jax.numpy as jnp
from jax import lax
from jax.experimental import pallas as pl
from jax.experimental.pallas import tpu as pltpu
from jax.experimental.pallas import tpu_sc as plsc

_N = 10000
_NPAD = 10240
_E = 320000
_NC = 2
_NS = 16
_F = 32
_TR = 256
_NT = 40
_EPT = _NT * _TR
_NB = 8
_EPAD = _NC * _NS * _NT * _TR
_RPT = _NPAD // _NS
_PR = _NPAD * _F // 128
_VR = _N * _F // 128
_PR16 = _NPAD * 16 // 128
_VR16 = _N * 16 // 128


def _mesh():
    return plsc.VectorSubcoreMesh(core_axis_name="c", subcore_axis_name="s",
                                  num_cores=_NC, num_subcores=_NS)


@functools.cache
def _make_sc_agg(gather_p, fw=_F):
    assert gather_p or fw == 16
    scratch = [
        pltpu.VMEM((_EPT,), jnp.int32),
        pltpu.VMEM((_EPT,), jnp.int32),
        pltpu.VMEM((_NB, _TR, fw), jnp.float32),
        pltpu.VMEM_SHARED((_NPAD, fw), jnp.float32),
    ]
    if not gather_p:
        scratch += [pltpu.VMEM((_RPT, 16), jnp.float32),
                    pltpu.VMEM((_RPT, _F), jnp.float32)]
    scratch += [pltpu.SemaphoreType.DMA] * (2 * _NB)

    n_out = 2 if gather_p else 4
    @functools.partial(
        pl.kernel,
        out_type=tuple(
            jax.ShapeDtypeStruct((_NPAD, fw), jnp.float32)
            for _ in range(2)
        ) + tuple(
            jax.ShapeDtypeStruct((_NPAD, _F), jnp.float32)
            for _ in range(n_out - 2)
        ),
        mesh=_mesh(),
        compiler_params=pltpu.CompilerParams(use_tc_tiling_on_sc=False),
        scratch_types=scratch,
    )
    def agg(p_hbm, ei_hbm, pad_hbm, zrows_hbm, *outs_and_scratch):
        outs = outs_and_scratch[:n_out]
        out0, out1 = outs[0], outs[1]
        sidx, didx, rows, acc, *rest = outs_and_scratch[n_out:]
        if gather_p:
            sems = rest
        else:
            buf16, buf32 = rest[0], rest[1]
            sems = rest[2:]
        gsem = sems[0:_NB]
        ssem = sems[_NB:2 * _NB]
        cid = lax.axis_index("c")
        sid = lax.axis_index("s")
        wid = sid * _NC + cid
        r0 = pl.multiple_of(sid * _RPT, _RPT)
        pltpu.sync_copy(zrows_hbm.at[pl.ds(0, _RPT)], acc.at[pl.ds(r0, _RPT)])
        e0 = pl.multiple_of(wid * _EPT, _TR)
        real = _E - (_NC * _NS - 1) * _EPT
        last = _NC * _NS - 1

        @pl.when(wid < last)
        def _():
            if gather_p:
                pltpu.sync_copy(ei_hbm.at[pl.ds(e0, _EPT)], sidx)
            pltpu.sync_copy(ei_hbm.at[pl.ds(_E + e0, _EPT)], didx)

        @pl.when(wid == last)
        def _():
            if gather_p:
                pltpu.sync_copy(ei_hbm.at[pl.ds(last * _EPT, real)],
                                sidx.at[pl.ds(0, real)])
                pltpu.sync_copy(pad_hbm, sidx.at[pl.ds(real, _EPT - real)])
            pltpu.sync_copy(ei_hbm.at[pl.ds(_E + last * _EPT, real)],
                            didx.at[pl.ds(0, real)])
            pltpu.sync_copy(pad_hbm, didx.at[pl.ds(real, _EPT - real)])

        if not gather_p:
            pltpu.sync_copy(p_hbm, rows.at[0])
        plsc.subcore_barrier()

        def gather(c, k):
            pltpu.async_copy(p_hbm.at[sidx.at[pl.ds(c * _TR, _TR)]],
                             rows.at[k], gsem[k])

        def wait_gather(k):
            pltpu.make_async_copy(p_hbm.at[sidx.at[pl.ds(0, _TR)]],
                                  rows.at[k], gsem[k]).wait()

        def scat(c, k):
            kk = k if gather_p else 0
            pltpu.async_copy(rows.at[kk],
                             acc.at[didx.at[pl.ds(c * _TR, _TR)]],
                             ssem[k], add=True)

        def wait_scat(k):
            kk = k if gather_p else 0
            pltpu.make_async_copy(rows.at[kk],
                                  acc.at[didx.at[pl.ds(0, _TR)]],
                                  ssem[k]).wait()

        half = _NB // 2
        niter = _NT // _NB
        if gather_p:
            for k in range(half):
                gather(k, k)

        def body(j, _):
            c0 = _NB * j
            for k in range(_NB):
                if gather_p:
                    wait_gather(k)
                scat(c0 + k, k)
                m = (k + half) % _NB
                if k < half:
                    @pl.when(j > 0)
                    def _():
                        wait_scat(m)

                    if gather_p:
                        gather(c0 + k + half, m)
                else:
                    wait_scat(m)

                    if gather_p:
                        @pl.when(j < niter - 1)
                        def _():
                            gather(c0 + k + half, m)
            return ()

        lax.fori_loop(0, niter, body, ())
        for k in range(half, _NB):
            wait_scat(k)
        plsc.subcore_barrier()

        if gather_p:
            @pl.when(cid == 0)
            def _():
                pltpu.sync_copy(acc.at[pl.ds(r0, _RPT)],
                                out0.at[pl.ds(r0, _RPT)])

            @pl.when(cid == 1)
            def _():
                pltpu.sync_copy(acc.at[pl.ds(r0, _RPT)],
                                out1.at[pl.ds(r0, _RPT)])
        else:
            pltpu.sync_copy(acc.at[pl.ds(r0, _RPT)], buf16)

            def dup(r, _):
                v = buf16[r, :]
                buf32[r, 0:16] = v
                buf32[r, 16:32] = v
                return ()

            lax.fori_loop(0, _RPT, dup, ())

            @pl.when(cid == 0)
            def _():
                pltpu.sync_copy(buf16, out0.at[pl.ds(r0, _RPT)])
                pltpu.sync_copy(buf32, outs[2].at[pl.ds(r0, _RPT)])

            @pl.when(cid == 1)
            def _():
                pltpu.sync_copy(buf16, out1.at[pl.ds(r0, _RPT)])
                pltpu.sync_copy(buf32, outs[3].at[pl.ds(r0, _RPT)])

    return agg


def _tc_p1_dis(x2, W1bd8, dega16, degb16, dega32, degb32):
    def body(x_ref, w_ref, da16, db16, da32, db32, p_ref, d16_ref, d32_ref):
        deg16 = da16[...] + db16[...] + 1.0
        row16 = lax.broadcasted_iota(jnp.int32, (_PR16, 1), 0)
        dis16 = jnp.where(row16 < _VR16, lax.rsqrt(deg16), 0.0)
        deg32 = da32[...] + db32[...] + 1.0
        row32 = lax.broadcasted_iota(jnp.int32, (_PR, 1), 0)
        dis32 = jnp.where(row32 < _VR, lax.rsqrt(deg32), 0.0)
        p = jnp.dot(x_ref[...], w_ref[...], preferred_element_type=jnp.float32)
        p_ref[...] = p * dis16
        d16_ref[...] = dis16
        d32_ref[...] = dis32

    return pl.pallas_call(
        body,
        out_shape=[
            jax.ShapeDtypeStruct((_PR16, 128), jnp.float32),
            jax.ShapeDtypeStruct((_PR16, 128), jnp.float32),
            jax.ShapeDtypeStruct((_PR, 128), jnp.float32),
        ],
    )(x2, W1bd8, dega16, degb16, dega32, degb32)


def _tc_p2(sa, sb, p1, dis16, W2bd8, b1t8):
    def body(sa_ref, sb_ref, p_ref, d16_ref, w_ref, b_ref, out_ref):
        d = d16_ref[...]
        h = d * (sa_ref[...] + sb_ref[...] + p_ref[...]) + b_ref[...]
        h = jnp.maximum(h, 0.0) * d
        out_ref[...] = jnp.dot(h, w_ref[...],
                               preferred_element_type=jnp.float32)

    return pl.pallas_call(
        body,
        out_shape=jax.ShapeDtypeStruct((_PR16, 256), jnp.float32),
    )(sa, sb, p1, dis16, W2bd8, b1t8)


def _tc_next_p(sa, sb, p, dis, Wbd, btile, relu):
    def body(sa_ref, sb_ref, p_ref, dis_ref, w_ref, b_ref, out_ref):
        d = dis_ref[...]
        h = d * (sa_ref[...] + sb_ref[...] + p_ref[...]) + b_ref[...]
        if relu:
            h = jnp.maximum(h, 0.0)
        out_ref[...] = jnp.dot(h, w_ref[...],
                               preferred_element_type=jnp.float32) * d

    return pl.pallas_call(
        body,
        out_shape=jax.ShapeDtypeStruct((_PR, 128), jnp.float32),
    )(sa, sb, p, dis, Wbd, btile)


def _tc_head(sa, sb, p3, dis, b3t, gfeat, fc1_W, fc1_b, fc2_W, fc2_b,
             fcg_W, fcg_b, fc3_W, fc3_b, fc4_W, fc4_b, fcf_W, fcf_b):
    def body(sa_ref, sb_ref, p_ref, dis_ref, b3_ref, gf_ref,
             fc1w, fc1b, fc2w, fc2b, fcgw, fcgb, fc3w, fc3b,
             fc4w, fc4b, fcfw, fcfb, out_ref):
        d = dis_ref[...]
        h3 = d * (sa_ref[...] + sb_ref[...] + p_ref[...]) + b3_ref[...]
        row = lax.broadcasted_iota(jnp.int32, (_PR, 1), 0)
        h3 = jnp.where(row < _VR, h3, 0.0)
        cs = jnp.sum(h3, axis=0, keepdims=True)
        g = (cs[:, 0:32] + cs[:, 32:64] + cs[:, 64:96]
             + cs[:, 96:128]) * (1.0 / _N)
        g = jnp.dot(g, fc1w[...], preferred_element_type=jnp.float32) + fc1b[...]
        g = jnp.dot(g, fc2w[...], preferred_element_type=jnp.float32) + fc2b[...]
        gf = jnp.maximum(jnp.dot(gf_ref[...], fcgw[...],
                                 preferred_element_type=jnp.float32) + fcgb[...], 0.0)
        gf = jnp.maximum(jnp.dot(gf, fc3w[...],
                                 preferred_element_type=jnp.float32) + fc3b[...], 0.0)
        gf = jnp.maximum(jnp.dot(gf, fc4w[...],
                                 preferred_element_type=jnp.float32) + fc4b[...], 0.0)
        top = jnp.dot(g, fcfw[0:128, :], preferred_element_type=jnp.float32)
        bot = jnp.dot(gf, fcfw[128:384, :], preferred_element_type=jnp.float32)
        out_ref[...] = jnp.maximum(top + bot + fcfb[...], 0.0)

    return pl.pallas_call(
        body,
        out_shape=jax.ShapeDtypeStruct((512, 256), jnp.float32),
    )(sa, sb, p3, dis, b3t, gfeat, fc1_W, fc1_b, fc2_W, fc2_b,
      fcg_W, fcg_b, fc3_W, fc3_b, fc4_W, fc4_b, fcf_W, fcf_b)


def _packed(a):
    return a.reshape(_PR, 128)


def _packed16(a):
    return a.reshape(_PR16, 128)


def kernel(x, edge_index, global_features, W1, b1, W2, b2, W3, b3,
           fc1_W, fc1_b, fc2_W, fc2_b, fcg_W, fcg_b, fc3_W, fc3_b,
           fc4_W, fc4_b, fcf_W, fcf_b):
    pad_e = _EPT - (_E - (_NC * _NS - 1) * _EPT)
    pad_rows = _N + (jnp.arange(pad_e, dtype=jnp.int32) % (_NPAD - _N))
    ei = edge_index.reshape(2 * _E)
    x2 = jnp.concatenate(
        [x, jnp.zeros((_NPAD - _N, 128), jnp.float32)], axis=0
    ).reshape(_NPAD // 8, 1024)
    ones16 = jnp.ones((_TR, 16), jnp.float32)
    z16 = jnp.zeros((_RPT, 16), jnp.float32)
    z32 = jnp.zeros((_RPT, _F), jnp.float32)
    eye4 = jnp.eye(4, dtype=jnp.float32)
    W1bd8 = jnp.kron(jnp.eye(8, dtype=jnp.float32), W1)
    W2bd8 = jnp.kron(jnp.eye(8, dtype=jnp.float32), W2)
    W3bd = jnp.kron(eye4, W3)
    b1t8 = jnp.tile(b1, 8).reshape(1, 128)
    b2t = jnp.tile(b2, 4).reshape(1, 128)
    b3t = jnp.tile(b3, 4).reshape(1, 128)

    agg32 = _make_sc_agg(True, _F)
    agg16 = _make_sc_agg(True, 16)
    deg = _make_sc_agg(False, 16)
    dega16, degb16, dega32, degb32 = deg(ones16, ei, pad_rows, z16)
    p1, dis16, dis32 = _tc_p1_dis(x2, W1bd8, _packed16(dega16),
                                  _packed16(degb16), _packed(dega32),
                                  _packed(degb32))
    s1a, s1b = agg16(p1.reshape(_NPAD, 16), ei, pad_rows, z16)
    p2 = _tc_p2(_packed16(s1a), _packed16(s1b), p1, dis16,
                W2bd8, b1t8).reshape(_NPAD, _F)
    s2a, s2b = agg32(p2, ei, pad_rows, z32)
    p3 = _tc_next_p(_packed(s2a), _packed(s2b), _packed(p2), dis32,
                    W3bd, b2t, False)
    s3a, s3b = agg32(p3.reshape(_NPAD, _F), ei, pad_rows, z32)
    return _tc_head(_packed(s3a), _packed(s3b), p3, dis32, b3t,
                    global_features,
                    fc1_W, fc1_b.reshape(1, 64), fc2_W, fc2_b.reshape(1, 128),
                    fcg_W, fcg_b.reshape(1, 64), fc3_W, fc3_b.reshape(1, 128),
                    fc4_W, fc4_b.reshape(1, 256), fcf_W, fcf_b.reshape(1, 256))
